# baseline (speedup 1.0000x reference)
import jax
import jax.numpy as jnp
from jax.experimental import pallas as pl

N, F, D = 524288, 26, 4
NF = N * F
ROWS = NF // 128
BLOCK = 512


def _expand4(v, prec):
    src = jax.lax.broadcasted_iota(jnp.int32, (128, 512), 1) // D
    lane = jax.lax.broadcasted_iota(jnp.int32, (128, 512), 0)
    expand = (src == lane).astype(jnp.float32)
    return jax.lax.dot_general(
        v, expand, (((1,), (0,)), ((), ())),
        preferred_element_type=jnp.float32, precision=prec)


def _lvr_block(x_ref, idx_ref, o_ref):
    xe = _expand4(x_ref[...], jax.lax.Precision.HIGHEST)
    ie = _expand4(idx_ref[...].astype(jnp.float32),
                  jax.lax.Precision.DEFAULT)
    k = (jax.lax.broadcasted_iota(jnp.int32, xe.shape, 1) % D
         ).astype(jnp.float32)
    o_ref[...] = jnp.where(k < ie, 1.0, jnp.where(k > ie, 0.0, xe))


def kernel(x, indices):
    out = pl.pallas_call(
        _lvr_block,
        grid=(ROWS // BLOCK,),
        in_specs=[
            pl.BlockSpec((BLOCK, 128), lambda i: (i, 0)),
            pl.BlockSpec((BLOCK, 128), lambda i: (i, 0)),
        ],
        out_specs=pl.BlockSpec((BLOCK, 512), lambda i: (i, 0)),
        out_shape=jax.ShapeDtypeStruct((ROWS, 512), jnp.float32),
    )(x.reshape(ROWS, 128), indices.reshape(ROWS, 128))
    return out.reshape(N, F, D)


# speedup vs baseline: 61.2615x; 61.2615x over previous
import jax
import jax.numpy as jnp
from jax.experimental import pallas as pl
from jax.experimental.pallas import tpu as pltpu

N, F, D = 524288, 26, 4
LANES = 128
NT = N // LANES
BNT = 32
BN = BNT * LANES


def _lvr_block(x_ref, idx_ref, o_ref):
    x3 = x_ref[...].reshape(F, BNT, LANES)
    i3 = idx_ref[...].reshape(F, BNT, LANES)
    for k in range(D):
        plane = jnp.where(i3 > k, 1.0, jnp.where(i3 < k, 0.0, x3))
        o_ref[:, pl.Slice(k, BNT, D), :] = plane


def kernel(x, indices):
    out = pl.pallas_call(
        _lvr_block,
        grid=(NT // BNT,),
        in_specs=[
            pl.BlockSpec((F, BN), lambda i: (0, i)),
            pl.BlockSpec((F, BN), lambda i: (0, i)),
        ],
        out_specs=pl.BlockSpec((F, D * BNT, LANES), lambda i: (0, i, 0)),
        out_shape=jax.ShapeDtypeStruct((F, D * NT, LANES), jnp.float32),
        compiler_params=pltpu.CompilerParams(
            dimension_semantics=("arbitrary",)),
    )(x.T, indices.T)
    return out.reshape(F, NT, D, LANES).transpose(1, 3, 0, 2).reshape(N, F, D)


# speedup vs baseline: 61.3419x; 1.0013x over previous
import jax
import jax.numpy as jnp
from jax.experimental import pallas as pl
from jax.experimental.pallas import tpu as pltpu

N, F, D = 524288, 26, 4
LANES = 128
NT = N // LANES
BNT = 32
BN = BNT * LANES


def _lvr_block(x_ref, idx_ref, o_ref):
    x3 = x_ref[...].reshape(F, BNT, LANES)
    i3 = idx_ref[...].reshape(F, BNT, LANES)
    for k in range(D):
        plane = jnp.where(i3 > k, 1.0, jnp.where(i3 < k, 0.0, x3))
        o_ref[:, pl.Slice(k, BNT, D), :] = plane


def kernel(x, indices):
    out = pl.pallas_call(
        _lvr_block,
        grid=(NT // BNT,),
        in_specs=[
            pl.BlockSpec((F, BN), lambda i: (0, i)),
            pl.BlockSpec((F, BN), lambda i: (0, i)),
        ],
        out_specs=pl.BlockSpec((F, D * BNT, LANES), lambda i: (0, i, 0)),
        out_shape=jax.ShapeDtypeStruct((F, D * NT, LANES), jnp.float32),
        compiler_params=pltpu.CompilerParams(
            dimension_semantics=("parallel",)),
    )(x.T, indices.T)
    return out.reshape(F, NT, D, LANES).transpose(1, 3, 0, 2).reshape(N, F, D)


# speedup vs baseline: 77.7542x; 1.2676x over previous
import jax
import jax.numpy as jnp
from jax.experimental import pallas as pl
from jax.experimental.pallas import tpu as pltpu

N, F, D = 524288, 26, 4
LANES = 128
NT = N // LANES
BNT = 64
BN = BNT * LANES


def _lvr_block(x_ref, idx_ref, o_ref):
    x3 = x_ref[...].reshape(F, BNT, LANES)
    i3 = idx_ref[...].reshape(F, BNT, LANES)
    for k in range(D):
        plane = jnp.where(i3 > k, 1.0, jnp.where(i3 < k, 0.0, x3))
        o_ref[:, pl.Slice(k, BNT, D), :] = plane


def kernel(x, indices):
    out = pl.pallas_call(
        _lvr_block,
        grid=(NT // BNT,),
        in_specs=[
            pl.BlockSpec((F, BN), lambda i: (0, i)),
            pl.BlockSpec((F, BN), lambda i: (0, i)),
        ],
        out_specs=pl.BlockSpec((F, D * BNT, LANES), lambda i: (0, i, 0)),
        out_shape=jax.ShapeDtypeStruct((F, D * NT, LANES), jnp.float32),
        compiler_params=pltpu.CompilerParams(
            dimension_semantics=("parallel",)),
    )(x.T, indices.T)
    return out.reshape(F, NT, D, LANES).transpose(1, 3, 0, 2).reshape(N, F, D)


# speedup vs baseline: 85.5747x; 1.1006x over previous
import jax
import jax.numpy as jnp
from jax.experimental import pallas as pl
from jax.experimental.pallas import tpu as pltpu

N, F, D = 524288, 26, 4
LANES = 128
NT = N // LANES
BNT = 128
BN = BNT * LANES


def _lvr_block(x_ref, idx_ref, o_ref):
    x3 = x_ref[...].reshape(F, BNT, LANES)
    i3 = idx_ref[...].reshape(F, BNT, LANES)
    for k in range(D):
        plane = jnp.where(i3 > k, 1.0, jnp.where(i3 < k, 0.0, x3))
        o_ref[:, pl.Slice(k, BNT, D), :] = plane


def kernel(x, indices):
    out = pl.pallas_call(
        _lvr_block,
        grid=(NT // BNT,),
        in_specs=[
            pl.BlockSpec((F, BN), lambda i: (0, i)),
            pl.BlockSpec((F, BN), lambda i: (0, i)),
        ],
        out_specs=pl.BlockSpec((F, D * BNT, LANES), lambda i: (0, i, 0)),
        out_shape=jax.ShapeDtypeStruct((F, D * NT, LANES), jnp.float32),
        compiler_params=pltpu.CompilerParams(
            dimension_semantics=("parallel",)),
    )(x.T, indices.T)
    return out.reshape(F, NT, D, LANES).transpose(1, 3, 0, 2).reshape(N, F, D)


# speedup vs baseline: 88.0329x; 1.0287x over previous
import jax
import jax.numpy as jnp
from jax.experimental import pallas as pl
from jax.experimental.pallas import tpu as pltpu

N, F, D = 524288, 26, 4
LANES = 128
NT = N // LANES
BNT = 256
BN = BNT * LANES


def _lvr_block(x_ref, idx_ref, o_ref):
    x3 = x_ref[...].reshape(F, BNT, LANES)
    i3 = idx_ref[...].reshape(F, BNT, LANES)
    for k in range(D):
        plane = jnp.where(i3 > k, 1.0, jnp.where(i3 < k, 0.0, x3))
        o_ref[:, pl.Slice(k, BNT, D), :] = plane


def kernel(x, indices):
    out = pl.pallas_call(
        _lvr_block,
        grid=(NT // BNT,),
        in_specs=[
            pl.BlockSpec((F, BN), lambda i: (0, i)),
            pl.BlockSpec((F, BN), lambda i: (0, i)),
        ],
        out_specs=pl.BlockSpec((F, D * BNT, LANES), lambda i: (0, i, 0)),
        out_shape=jax.ShapeDtypeStruct((F, D * NT, LANES), jnp.float32),
        compiler_params=pltpu.CompilerParams(
            dimension_semantics=("parallel",)),
    )(x.T, indices.T)
    return out.reshape(F, NT, D, LANES).transpose(1, 3, 0, 2).reshape(N, F, D)


# speedup vs baseline: 89.4950x; 1.0166x over previous
import jax
import jax.numpy as jnp
from jax.experimental import pallas as pl
from jax.experimental.pallas import tpu as pltpu

N, F, D = 524288, 26, 4
LANES = 128
NT = N // LANES
BNT = 256
BN = BNT * LANES


CH = 8


def _lvr_block(x_ref, idx_ref, o_ref):
    def body(c, carry):
        sl = pl.ds(c * CH * LANES, CH * LANES)
        x3 = x_ref[:, sl].reshape(F, CH, LANES)
        i3 = idx_ref[:, sl].reshape(F, CH, LANES)
        base = c * CH * D
        for k in range(D):
            if k == 0:
                plane = jnp.where(i3 > 0, 1.0, x3)
            elif k == D - 1:
                plane = jnp.where(i3 < D - 1, 0.0, x3)
            else:
                plane = jnp.where(i3 > k, 1.0, jnp.where(i3 < k, 0.0, x3))
            o_ref[:, pl.Slice(base + k, CH, D), :] = plane
        return carry
    jax.lax.fori_loop(0, BNT // CH, body, 0)


def kernel(x, indices):
    out = pl.pallas_call(
        _lvr_block,
        grid=(NT // BNT,),
        in_specs=[
            pl.BlockSpec((F, BN), lambda i: (0, i)),
            pl.BlockSpec((F, BN), lambda i: (0, i)),
        ],
        out_specs=pl.BlockSpec((F, D * BNT, LANES), lambda i: (0, i, 0)),
        out_shape=jax.ShapeDtypeStruct((F, D * NT, LANES), jnp.float32),
        compiler_params=pltpu.CompilerParams(
            dimension_semantics=("parallel",)),
    )(x.T, indices.T)
    return out.reshape(F, NT, D, LANES).transpose(1, 3, 0, 2).reshape(N, F, D)
